# baseline (speedup 1.0000x reference)
import functools

import jax
import jax.numpy as jnp
import numpy as np
from jax import lax
from jax.experimental import pallas as pl
from jax.experimental.pallas import tpu as pltpu
from jax.experimental.pallas import tpu_sc as plsc

_N = 1_000_000
_M = 500_000
_D = 32
_NC = 2
_NS = 16
_W = _NC * _NS
_B = _N // _W
_C = 1250
_NCHUNK = _B // _C
_RPT = _C + 8


def _build_routing():
    index = np.asarray(
        jax.random.permutation(jax.random.key(42), _N)[:_M]
    ).astype(np.int32)
    order = np.argsort(index, kind="stable").astype(np.int32)
    dst_sorted = index[order]
    bounds = np.searchsorted(dst_sorted, np.arange(0, _N + _C, _C))
    counts = np.diff(bounds)
    groups = int(np.ceil(counts.max() / 128))
    k = groups * 128
    loc = np.empty((_W * _NCHUNK, k), dtype=np.int32)
    src = np.zeros((_W * _NCHUNK, k), dtype=np.int32)
    for t in range(_W * _NCHUNK):
        w = t // _NCHUNK
        sid = w // _NC
        base = sid * _RPT
        loc[t] = base + _C
        s, e = bounds[t], bounds[t + 1]
        n = e - s
        loc[t, :n] = (dst_sorted[s:e] - t * _C) + base
        src[t, :n] = order[s:e]
    loc = loc.reshape(_W, _NCHUNK, groups, 128)
    src = src.reshape(_W, _NCHUNK, groups, 128)
    return index, loc, src, groups


_INDEX_NP, _LOC_NP, _SRC_NP, _G = _build_routing()
_INDEX = jnp.asarray(_INDEX_NP)
_LOC = jnp.asarray(_LOC_NP)
_SRC = jnp.asarray(_SRC_NP)

_mesh = plsc.VectorSubcoreMesh(
    core_axis_name="c", subcore_axis_name="s", num_cores=_NC, num_subcores=_NS
)


@functools.partial(
    pl.kernel,
    out_type=jax.ShapeDtypeStruct((_N, _D), jnp.float32),
    mesh=_mesh,
    compiler_params=pltpu.CompilerParams(use_tc_tiling_on_sc=False),
    scratch_types=[
        pltpu.VMEM_SHARED((_NS * _RPT, _D), jnp.float32),
        pltpu.VMEM((_G, 128, _D), jnp.float32),
        pltpu.VMEM((_G, 128), jnp.int32),
        pltpu.VMEM((_G, 128), jnp.int32),
        pltpu.SemaphoreType.DMA,
        pltpu.SemaphoreType.DMA,
    ],
)
def _sc_index_add(x_hbm, y_hbm, loc_hbm, src_hbm, out_hbm,
                  xsh, yv, locv, srcv, sem_y, sem_x):
    cid = lax.axis_index("c")
    sid = lax.axis_index("s")
    wid = sid * _NC + cid
    tile_base = sid * _RPT

    def chunk_body(c, carry):
        row0 = wid * _B + c * _C
        cp_x = pltpu.async_copy(
            x_hbm.at[pl.ds(row0, _C)],
            xsh.at[pl.ds(tile_base, _C)],
            sem_x,
        )
        pltpu.sync_copy(loc_hbm.at[wid, c], locv)
        pltpu.sync_copy(src_hbm.at[wid, c], srcv)
        gathers = [
            pltpu.async_copy(y_hbm.at[srcv.at[g]], yv.at[g], sem_y)
            for g in range(_G)
        ]
        for g in gathers:
            g.wait()
        cp_x.wait()
        for g in range(_G):
            pltpu.sync_copy(yv.at[g], xsh.at[locv.at[g]], add=True)
        pltpu.sync_copy(
            xsh.at[pl.ds(tile_base, _C)],
            out_hbm.at[pl.ds(row0, _C)],
        )
        return carry

    lax.fori_loop(0, _NCHUNK, chunk_body, 0)


def kernel(x, y):
    out = _sc_index_add(x, y, _LOC, _SRC)
    return out, _INDEX


# speedup vs baseline: 1.2532x; 1.2532x over previous
import functools

import jax
import jax.numpy as jnp
import numpy as np
from jax import lax
from jax.experimental import pallas as pl
from jax.experimental.pallas import tpu as pltpu
from jax.experimental.pallas import tpu_sc as plsc

_N = 1_000_000
_M = 500_000
_D = 32
_NC = 2
_NS = 16
_W = _NC * _NS
_B = _N // _W
_C = 1250
_NCHUNK = _B // _C
_RPT = _C + 8


def _build_routing():
    index = np.asarray(
        jax.random.permutation(jax.random.key(42), _N)[:_M]
    ).astype(np.int32)
    order = np.argsort(index, kind="stable").astype(np.int32)
    dst_sorted = index[order]
    bounds = np.searchsorted(dst_sorted, np.arange(0, _N + _C, _C))
    counts = np.diff(bounds)
    groups = int(np.ceil(counts.max() / 64))
    k = groups * 64
    loc = np.empty((_W * _NCHUNK, k), dtype=np.int32)
    src = np.zeros((_W * _NCHUNK, k), dtype=np.int32)
    for t in range(_W * _NCHUNK):
        w = t // _NCHUNK
        sid = w // _NC
        base = sid * _RPT
        loc[t] = base + _C
        s, e = bounds[t], bounds[t + 1]
        n = e - s
        loc[t, :n] = (dst_sorted[s:e] - t * _C) + base
        src[t, :n] = order[s:e]
    loc = loc.reshape(_W, _NCHUNK, groups, 64)
    src = src.reshape(_W, _NCHUNK, groups, 64)
    return index, loc, src, groups


_INDEX_NP, _LOC_NP, _SRC_NP, _G = _build_routing()
_INDEX = jnp.asarray(_INDEX_NP)
_LOC = jnp.asarray(_LOC_NP)
_SRC = jnp.asarray(_SRC_NP)

_mesh = plsc.VectorSubcoreMesh(
    core_axis_name="c", subcore_axis_name="s", num_cores=_NC, num_subcores=_NS
)


@functools.partial(
    pl.kernel,
    out_type=jax.ShapeDtypeStruct((_N, _D), jnp.float32),
    mesh=_mesh,
    compiler_params=pltpu.CompilerParams(use_tc_tiling_on_sc=False),
    scratch_types=[
        pltpu.VMEM_SHARED((_NS * _RPT, _D), jnp.float32),
        pltpu.VMEM((_G, 64, _D), jnp.float32),
        pltpu.VMEM((_G, 64), jnp.int32),
        pltpu.VMEM((_G, 64), jnp.int32),
        pltpu.SemaphoreType.DMA,
        pltpu.SemaphoreType.DMA,
    ],
)
def _sc_index_add(x_hbm, y_hbm, loc_hbm, src_hbm, out_hbm,
                  xsh, yv, locv, srcv, sem_y, sem_x):
    cid = lax.axis_index("c")
    sid = lax.axis_index("s")
    wid = sid * _NC + cid
    tile_base = sid * _RPT

    def chunk_body(c, carry):
        row0 = wid * _B + c * _C
        cp_x = pltpu.async_copy(
            x_hbm.at[pl.ds(row0, _C)],
            xsh.at[pl.ds(tile_base, _C)],
            sem_x,
        )
        pltpu.sync_copy(loc_hbm.at[wid, c], locv)
        pltpu.sync_copy(src_hbm.at[wid, c], srcv)
        gathers = [
            pltpu.async_copy(y_hbm.at[srcv.at[g]], yv.at[g], sem_y)
            for g in range(_G)
        ]
        for g in gathers:
            g.wait()
        cp_x.wait()
        for g in range(_G):
            pltpu.sync_copy(yv.at[g], xsh.at[locv.at[g]], add=True)
        pltpu.sync_copy(
            xsh.at[pl.ds(tile_base, _C)],
            out_hbm.at[pl.ds(row0, _C)],
        )
        return carry

    lax.fori_loop(0, _NCHUNK, chunk_body, 0)


def kernel(x, y):
    out = _sc_index_add(x, y, _LOC, _SRC)
    return out, _INDEX
